# baseline (device time: 49860 ns/iter reference)
import jax
import jax.numpy as jnp
from jax import lax
from jax.experimental import pallas as pl
from jax.experimental.pallas import tpu as pltpu

N_DEV = 8
N_TOK = 1024
D_IN = 512
D_OUT = 1024
N_EXP = 32
CAP = 25
SLOTS_PER_EXP = 32
E_LOCAL = N_EXP // N_DEV
ROWS_PER_DEV = E_LOCAL * SLOTS_PER_EXP


def kernel(x, router_W, route_idx, expert_W):
    del router_W

    def body(x_ref, idx_ref, w_ref, out_ref, ag_ref, send_sems, recv_sems):
        me = lax.axis_index("i")
        left = lax.rem(me + N_DEV - 1, N_DEV)
        right = lax.rem(me + 1, N_DEV)

        e = idx_ref[:, :]
        cols_e = lax.broadcasted_iota(jnp.int32, (N_TOK, N_EXP), 1)
        onehot = (e == cols_e).astype(jnp.bfloat16)
        row_i = lax.broadcasted_iota(jnp.int32, (N_TOK, N_TOK), 0)
        col_i = lax.broadcasted_iota(jnp.int32, (N_TOK, N_TOK), 1)
        lower = (col_i < row_i).astype(jnp.bfloat16)
        m = lax.dot_general(
            lower, onehot, (((1,), (0,)), ((), ())),
            preferred_element_type=jnp.float32,
        )
        rank = jnp.sum(m * onehot.astype(jnp.float32), axis=1, keepdims=True)
        rank = rank.astype(jnp.int32)
        keep = rank < CAP
        gslot = jnp.where(keep, e * SLOTS_PER_EXP + rank, -1)

        xb = x_ref[:, :].astype(jnp.bfloat16)
        base = me * ROWS_PER_DEV
        cols_r = lax.broadcasted_iota(jnp.int32, (N_TOK, ROWS_PER_DEV), 1)
        pmine = (gslot == cols_r + base).astype(jnp.bfloat16)
        cx = lax.dot_general(
            pmine, xb, (((0,), (0,)), ((), ())),
            preferred_element_type=jnp.float32,
        ).astype(jnp.bfloat16)
        for j in range(E_LOCAL):
            wj = w_ref[j, :, :].astype(jnp.bfloat16)
            oj = jnp.dot(
                cx[j * SLOTS_PER_EXP:(j + 1) * SLOTS_PER_EXP, :], wj,
                preferred_element_type=jnp.float32,
            )
            ag_ref[pl.ds(base + j * SLOTS_PER_EXP, SLOTS_PER_EXP), :] = (
                oj.astype(jnp.bfloat16)
            )

        barrier_sem = pltpu.get_barrier_semaphore()
        pl.semaphore_signal(
            barrier_sem, inc=1, device_id=(left,),
            device_id_type=pl.DeviceIdType.MESH,
        )
        pl.semaphore_signal(
            barrier_sem, inc=1, device_id=(right,),
            device_id_type=pl.DeviceIdType.MESH,
        )
        pl.semaphore_wait(barrier_sem, 2)

        for h in range(N_DEV - 1):
            origin = lax.rem(me + N_DEV - h, N_DEV)
            rows = pl.ds(origin * ROWS_PER_DEV, ROWS_PER_DEV)
            rdma = pltpu.make_async_remote_copy(
                src_ref=ag_ref.at[rows, :],
                dst_ref=ag_ref.at[rows, :],
                send_sem=send_sems.at[h],
                recv_sem=recv_sems.at[h],
                device_id=(right,),
                device_id_type=pl.DeviceIdType.MESH,
            )
            rdma.start()
            rdma.wait()

        cols_g = lax.broadcasted_iota(jnp.int32, (N_TOK, N_TOK), 1)
        perm = (gslot == cols_g).astype(jnp.bfloat16)
        out_ref[:, :] = lax.dot_general(
            perm, ag_ref[:, :], (((1,), (0,)), ((), ())),
            preferred_element_type=jnp.float32,
        )

    return pl.pallas_call(
        body,
        out_shape=jax.ShapeDtypeStruct((N_TOK, D_OUT), jnp.float32),
        in_specs=[
            pl.BlockSpec(memory_space=pltpu.VMEM),
            pl.BlockSpec(memory_space=pltpu.VMEM),
            pl.BlockSpec(memory_space=pltpu.VMEM),
        ],
        out_specs=pl.BlockSpec(memory_space=pltpu.VMEM),
        scratch_shapes=[
            pltpu.VMEM((N_TOK, D_OUT), jnp.bfloat16),
            pltpu.SemaphoreType.DMA((N_DEV - 1,)),
            pltpu.SemaphoreType.DMA((N_DEV - 1,)),
        ],
        compiler_params=pltpu.CompilerParams(collective_id=0),
    )(x, route_idx, expert_W)


# device time: 34686 ns/iter; 1.4375x vs baseline; 1.4375x over previous
import jax
import jax.numpy as jnp
from jax import lax
from jax.experimental import pallas as pl
from jax.experimental.pallas import tpu as pltpu

N_DEV = 8
N_TOK = 1024
D_IN = 512
D_OUT = 1024
N_EXP = 32
CAP = 25
SLOTS_PER_EXP = 32
E_LOCAL = N_EXP // N_DEV
ROWS_PER_DEV = E_LOCAL * SLOTS_PER_EXP


def kernel(x, router_W, route_idx, expert_W):
    del router_W

    def body(x_ref, idx_ref, w_ref, out_ref, ag_ref, send_sems, recv_sems):
        me = lax.axis_index("i")

        e = idx_ref[:, :]
        cols_e = lax.broadcasted_iota(jnp.int32, (N_TOK, N_EXP), 1)
        onehot = (e == cols_e).astype(jnp.bfloat16)
        row_i = lax.broadcasted_iota(jnp.int32, (N_TOK, N_TOK), 0)
        col_i = lax.broadcasted_iota(jnp.int32, (N_TOK, N_TOK), 1)
        lower = (col_i < row_i).astype(jnp.bfloat16)
        m = lax.dot_general(
            lower, onehot, (((1,), (0,)), ((), ())),
            preferred_element_type=jnp.float32,
        )
        rank = jnp.sum(m * onehot.astype(jnp.float32), axis=1, keepdims=True)
        rank = rank.astype(jnp.int32)
        keep = rank < CAP
        gslot = jnp.where(keep, e * SLOTS_PER_EXP + rank, -1)

        xb = x_ref[:, :].astype(jnp.bfloat16)
        base = me * ROWS_PER_DEV
        cols_r = lax.broadcasted_iota(jnp.int32, (N_TOK, ROWS_PER_DEV), 1)
        pmine = (gslot == cols_r + base).astype(jnp.bfloat16)
        cx = lax.dot_general(
            pmine, xb, (((0,), (0,)), ((), ())),
            preferred_element_type=jnp.float32,
        ).astype(jnp.bfloat16)
        for j in range(E_LOCAL):
            wj = w_ref[j, :, :].astype(jnp.bfloat16)
            oj = jnp.dot(
                cx[j * SLOTS_PER_EXP:(j + 1) * SLOTS_PER_EXP, :], wj,
                preferred_element_type=jnp.float32,
            )
            ag_ref[pl.ds(base + j * SLOTS_PER_EXP, SLOTS_PER_EXP), :] = (
                oj.astype(jnp.bfloat16)
            )

        barrier_sem = pltpu.get_barrier_semaphore()
        for k in range(1, N_DEV):
            peer = lax.rem(me + k, N_DEV)
            pl.semaphore_signal(
                barrier_sem, inc=1, device_id=(peer,),
                device_id_type=pl.DeviceIdType.MESH,
            )
        pl.semaphore_wait(barrier_sem, N_DEV - 1)

        my_rows = pl.ds(base, ROWS_PER_DEV)
        sends = []
        for k in range(1, N_DEV):
            peer = lax.rem(me + k, N_DEV)
            rdma = pltpu.make_async_remote_copy(
                src_ref=ag_ref.at[my_rows, :],
                dst_ref=ag_ref.at[my_rows, :],
                send_sem=send_sems.at[k - 1],
                recv_sem=recv_sems.at[k - 1],
                device_id=(peer,),
                device_id_type=pl.DeviceIdType.MESH,
            )
            rdma.start()
            sends.append(rdma)

        cols_g = lax.broadcasted_iota(jnp.int32, (N_TOK, N_TOK), 1)
        perm = (gslot == cols_g).astype(jnp.bfloat16)

        for k in range(1, N_DEV):
            src = lax.rem(me + N_DEV - k, N_DEV)
            rows = pl.ds(src * ROWS_PER_DEV, ROWS_PER_DEV)
            recv = pltpu.make_async_remote_copy(
                src_ref=ag_ref.at[rows, :],
                dst_ref=ag_ref.at[rows, :],
                send_sem=send_sems.at[k - 1],
                recv_sem=recv_sems.at[k - 1],
                device_id=(src,),
                device_id_type=pl.DeviceIdType.MESH,
            )
            recv.wait_recv()
        for rdma in sends:
            rdma.wait_send()

        out_ref[:, :] = lax.dot_general(
            perm, ag_ref[:, :], (((1,), (0,)), ((), ())),
            preferred_element_type=jnp.float32,
        )

    return pl.pallas_call(
        body,
        out_shape=jax.ShapeDtypeStruct((N_TOK, D_OUT), jnp.float32),
        in_specs=[
            pl.BlockSpec(memory_space=pltpu.VMEM),
            pl.BlockSpec(memory_space=pltpu.VMEM),
            pl.BlockSpec(memory_space=pltpu.VMEM),
        ],
        out_specs=pl.BlockSpec(memory_space=pltpu.VMEM),
        scratch_shapes=[
            pltpu.VMEM((N_TOK, D_OUT), jnp.bfloat16),
            pltpu.SemaphoreType.DMA((N_DEV - 1,)),
            pltpu.SemaphoreType.DMA((N_DEV - 1,)),
        ],
        compiler_params=pltpu.CompilerParams(collective_id=0),
    )(x, route_idx, expert_W)
